# Initial kernel scaffold; baseline (speedup 1.0000x reference)
import jax
import jax.numpy as jnp
from jax.experimental import pallas as pl


def kernel(x, edge_index, edge_attr, batch, W1_rel, W1_root, b1, W2_rel, W2_root, b2, c1w, c1b, c2w, c2b, c3w, c3b, m1w, m1b, m2w, m2b, m3w, m3b):
    raise NotImplementedError("write your pallas kernel here")


# speedup vs baseline: 5.8965x; 5.8965x over previous
import functools

import numpy as np
import jax
import jax.numpy as jnp
from jax import lax
from jax.experimental import pallas as pl
from jax.experimental.pallas import tpu as pltpu
from jax.experimental.pallas import tpu_sc as plsc

N_NODES = 12288
N_EDGES = 98304
NC = 2
NS = 16
CHUNK = 128
EDGES_PER_TILE = N_EDGES // NS
CHUNKS_PER_TILE = EDGES_PER_TILE // CHUNK
IDX_ROWS = N_EDGES // CHUNK


def _make_edge_kernel(feat):
    half = feat // 2
    nvec = half // 16
    rows_per_tile = N_NODES // NS

    mesh = plsc.VectorSubcoreMesh(core_axis_name="c", subcore_axis_name="s",
                                  num_cores=NC)

    @functools.partial(
        pl.kernel,
        mesh=mesh,
        compiler_params=pltpu.CompilerParams(use_tc_tiling_on_sc=False),
        out_type=[jax.ShapeDtypeStruct((N_NODES, half), jnp.float32)] * 2,
        scratch_types=[
            pltpu.VMEM((CHUNKS_PER_TILE, CHUNK), jnp.int32),
            pltpu.VMEM((CHUNKS_PER_TILE, CHUNK), jnp.int32),
            pltpu.VMEM((CHUNK, 16), jnp.float32),
            pltpu.VMEM((CHUNK, half), jnp.float32),
            pltpu.VMEM_SHARED((N_NODES, half), jnp.float32),
            pltpu.SemaphoreType.DMA,
        ],
    )
    def edge_kernel(tab_a, tab_b, src2, dst2, ew3, out_a, out_b,
                    src_v, dst_v, ew_v, rows_v, acc, sem):
        c = lax.axis_index("c")
        s = lax.axis_index("s")

        zero16 = jnp.zeros((16,), jnp.float32)

        def zbody(i, carry):
            for j in range(nvec):
                rows_v[i, pl.ds(j * 16, 16)] = zero16
            return carry

        lax.fori_loop(0, CHUNK, zbody, 0)
        for r in range(rows_per_tile // CHUNK):
            pltpu.sync_copy(rows_v,
                            acc.at[pl.ds(s * rows_per_tile + r * CHUNK, CHUNK)])

        base = s * CHUNKS_PER_TILE
        pltpu.sync_copy(src2.at[pl.ds(base, CHUNKS_PER_TILE)], src_v)
        pltpu.sync_copy(dst2.at[pl.ds(base, CHUNKS_PER_TILE)], dst_v)
        plsc.subcore_barrier()

        def run_core(tab):
            def chunk_body(t, carry):
                gather = pltpu.async_copy(tab.at[src_v.at[t]], rows_v, sem)
                pltpu.sync_copy(ew3.at[base + t], ew_v)
                gather.wait()

                def scale_body(k, carry2):
                    w = ew_v[k, :]
                    for j in range(nvec):
                        rows_v[k, pl.ds(j * 16, 16)] = (
                            rows_v[k, pl.ds(j * 16, 16)] * w)
                    return carry2

                lax.fori_loop(0, CHUNK, scale_body, 0)

                pltpu.sync_copy(rows_v, acc.at[dst_v.at[t]], add=True)
                return carry

            lax.fori_loop(0, CHUNKS_PER_TILE, chunk_body, 0)

        pl.when(c == 0)(lambda: run_core(tab_a))
        pl.when(c == 1)(lambda: run_core(tab_b))
        plsc.subcore_barrier()

        def writeback(out):
            for r in range(rows_per_tile // CHUNK):
                off = s * rows_per_tile + r * CHUNK
                pltpu.sync_copy(acc.at[pl.ds(off, CHUNK)],
                                out.at[pl.ds(off, CHUNK)])

        pl.when(c == 0)(lambda: writeback(out_a))
        pl.when(c == 1)(lambda: writeback(out_b))

    return edge_kernel


_edge_kernel_cache = {}


def _edge_kernel(feat):
    if feat not in _edge_kernel_cache:
        _edge_kernel_cache[feat] = _make_edge_kernel(feat)
    return _edge_kernel_cache[feat]


def _proj1_body(h_ref, w_ref, b_ref, rel_a_ref, rel_b_ref, root_ref):
    p = jnp.dot(h_ref[...], w_ref[...], preferred_element_type=jnp.float32)
    rel_a_ref[...] = p[:, :64]
    rel_b_ref[...] = p[:, 64:128]
    root_ref[...] = p[:, 128:] + b_ref[...]


_proj1 = pl.pallas_call(
    _proj1_body,
    grid=(12,),
    in_specs=[
        pl.BlockSpec((1024, 672), lambda i: (i, 0)),
        pl.BlockSpec((672, 256), lambda i: (0, 0)),
        pl.BlockSpec((1, 128), lambda i: (0, 0)),
    ],
    out_specs=[
        pl.BlockSpec((1024, 64), lambda i: (i, 0)),
        pl.BlockSpec((1024, 64), lambda i: (i, 0)),
        pl.BlockSpec((1024, 128), lambda i: (i, 0)),
    ],
    out_shape=[jax.ShapeDtypeStruct((N_NODES, 64), jnp.float32),
               jax.ShapeDtypeStruct((N_NODES, 64), jnp.float32),
               jax.ShapeDtypeStruct((N_NODES, 128), jnp.float32)],
)


def _proj2_body(agg_a_ref, agg_b_ref, root1_ref, w_ref, b_ref,
                rel_a_ref, rel_b_ref, root_ref):
    root1 = root1_ref[...]
    h1a = jnp.tanh(agg_a_ref[...] + root1[:, :64])
    h1b = jnp.tanh(agg_b_ref[...] + root1[:, 64:])
    p = (jnp.dot(h1a, w_ref[:64, :], preferred_element_type=jnp.float32)
         + jnp.dot(h1b, w_ref[64:, :], preferred_element_type=jnp.float32))
    rel_a_ref[...] = p[:, :32]
    rel_b_ref[...] = p[:, 32:64]
    root_ref[...] = p[:, 64:] + b_ref[...]


_proj2 = pl.pallas_call(
    _proj2_body,
    grid=(12,),
    in_specs=[
        pl.BlockSpec((1024, 64), lambda i: (i, 0)),
        pl.BlockSpec((1024, 64), lambda i: (i, 0)),
        pl.BlockSpec((1024, 128), lambda i: (i, 0)),
        pl.BlockSpec((128, 128), lambda i: (0, 0)),
        pl.BlockSpec((1, 64), lambda i: (0, 0)),
    ],
    out_specs=[
        pl.BlockSpec((1024, 32), lambda i: (i, 0)),
        pl.BlockSpec((1024, 32), lambda i: (i, 0)),
        pl.BlockSpec((1024, 64), lambda i: (i, 0)),
    ],
    out_shape=[jax.ShapeDtypeStruct((N_NODES, 32), jnp.float32),
               jax.ShapeDtypeStruct((N_NODES, 32), jnp.float32),
               jax.ShapeDtypeStruct((N_NODES, 64), jnp.float32)],
)


def _pool_body(agg_a_ref, agg_b_ref, root_ref, out_ref):
    root = root_ref[...]
    h2a = jnp.tanh(agg_a_ref[...] + root[:, :, :32])
    h2b = jnp.tanh(agg_b_ref[...] + root[:, :, 32:])
    out_ref[:, :32] = jnp.sum(h2a, axis=1) * (1.0 / 32.0)
    out_ref[:, 32:] = jnp.sum(h2b, axis=1) * (1.0 / 32.0)


_pool = pl.pallas_call(
    _pool_body,
    grid=(12,),
    in_specs=[
        pl.BlockSpec((32, 32, 32), lambda i: (i, 0, 0)),
        pl.BlockSpec((32, 32, 32), lambda i: (i, 0, 0)),
        pl.BlockSpec((32, 32, 64), lambda i: (i, 0, 0)),
    ],
    out_specs=pl.BlockSpec((32, 64), lambda i: (i, 0)),
    out_shape=jax.ShapeDtypeStruct((384, 64), jnp.float32),
)


def _head_body(z_ref, wc1_ref, bc1_ref, wc2_ref, bc2_ref, wc3_ref, bc3_ref,
               w4_ref, b4_ref, w5_ref, b5_ref, w6_ref, b6_ref, out_ref):
    z = z_ref[...]
    z = jnp.maximum(
        jnp.dot(z, wc1_ref[...], preferred_element_type=jnp.float32)
        + bc1_ref[...], 0.0)
    z = jnp.maximum(
        jnp.dot(z, wc2_ref[...], preferred_element_type=jnp.float32)
        + bc2_ref[...], 0.0)
    z = jnp.maximum(
        jnp.dot(z, wc3_ref[...], preferred_element_type=jnp.float32)
        + bc3_ref[...], 0.0)
    z = jnp.maximum(
        jnp.dot(z, w4_ref[...], preferred_element_type=jnp.float32)
        + b4_ref[...], 0.0)
    z = jnp.maximum(
        jnp.dot(z, w5_ref[...], preferred_element_type=jnp.float32)
        + b5_ref[...], 0.0)
    out_ref[...] = (
        jnp.dot(z, w6_ref[...], preferred_element_type=jnp.float32)
        + b6_ref[...])


def _head(z, *ws):
    return pl.pallas_call(
        _head_body,
        out_shape=jax.ShapeDtypeStruct((32, 1), jnp.float32),
    )(z, *ws)


def _shift_mats(n):
    s = np.zeros((2, n, n - 1), np.float32)
    for k in range(2):
        for b in range(n - 1):
            s[k, b + k, b] = 1.0
    return s


_S1 = _shift_mats(64)
_S2 = _shift_mats(63)
_S3 = _shift_mats(62)


def kernel(x, edge_index, edge_attr, batch,
           W1_rel, W1_root, b1, W2_rel, W2_root, b2,
           c1w, c1b, c2w, c2b, c3w, c3b,
           m1w, m1b, m2w, m2b, m3w, m3b):
    h = jnp.reshape(x, (-1, x.shape[-1]))

    src2 = edge_index[0].reshape(IDX_ROWS, CHUNK)
    dst2 = edge_index[1].reshape(IDX_ROWS, CHUNK)
    ew3 = jnp.broadcast_to(edge_attr[:, None],
                           (N_EDGES, 16)).reshape(IDX_ROWS, CHUNK, 16)

    w1cat = jnp.concatenate([W1_rel, W1_root], axis=1)
    rel1a, rel1b, root1 = _proj1(h, w1cat, b1[None, :])
    agg1a, agg1b = _edge_kernel(128)(rel1a, rel1b, src2, dst2, ew3)

    w2cat = jnp.concatenate([W2_rel, W2_root], axis=1)
    rel2a, rel2b, root2 = _proj2(agg1a, agg1b, root1, w2cat, b2[None, :])
    agg2a, agg2b = _edge_kernel(64)(rel2a, rel2b, src2, dst2, ew3)

    pooled = _pool(agg2a.reshape(384, 32, 32), agg2b.reshape(384, 32, 32),
                   root2.reshape(384, 32, 64))
    z = pooled.reshape(32, 768)

    wc1 = jnp.einsum("oik,kab->iaob", c1w, _S1).reshape(768, 504)
    wc2 = jnp.einsum("oik,kab->iaob", c2w, _S2).reshape(504, 248)
    wc3 = jnp.einsum("oik,kab->iaob", c3w, _S3).reshape(248, 61)
    bc1 = jnp.repeat(c1b, 63)[None, :]
    bc2 = jnp.repeat(c2b, 62)[None, :]
    bc3 = jnp.repeat(c3b, 61)[None, :]

    return _head(z, wc1, bc1, wc2, bc2, wc3, bc3,
                 m1w.T, m1b[None, :], m2w.T, m2b[None, :],
                 m3w.T, m3b[None, :])
